# Initial kernel scaffold; baseline (speedup 1.0000x reference)
import jax
import jax.numpy as jnp
from jax.experimental import pallas as pl


def kernel(x, edge_index, W1_0, b1_0, W2_0, b2_0, W1_1, b1_1, W2_1, b2_1):
    raise NotImplementedError("write your pallas kernel here")


# speedup vs baseline: 7.2760x; 7.2760x over previous
import functools

import jax
import jax.numpy as jnp
from jax import lax
from jax.experimental import pallas as pl
from jax.experimental.pallas import tpu as pltpu
from jax.experimental.pallas import tpu_sc as plsc

_N = 10000
_E = 320000
_D = 128
_NC = 2
_NS = 16
_NW = _NC * _NS
_EPW = _E // _NW
_CH = 80
_NCH = _EPW // _CH
_NP = 10240
_RPT = _NP // _NS

_BR = 1000


def _sc_segment_sum(h, src_r, dst_r, zeros):
    mesh = plsc.VectorSubcoreMesh(core_axis_name="c", subcore_axis_name="s")

    @functools.partial(
        pl.kernel,
        out_type=jax.ShapeDtypeStruct((_NC, _NP, _D), jnp.float32),
        mesh=mesh,
        scratch_types=[
            pltpu.VMEM((_NCH, _CH), jnp.int32),
            pltpu.VMEM((_NCH, _CH), jnp.int32),
            pltpu.VMEM((_CH, _D), jnp.float32),
            pltpu.VMEM_SHARED((_NP, _D), jnp.float32),
            pltpu.SemaphoreType.DMA,
        ],
    )
    def k(h_hbm, src_hbm, dst_hbm, zero_hbm, out_hbm,
          src_v, dst_v, rows_v, agg_s, sem):
        c = lax.axis_index("c")
        s = lax.axis_index("s")
        wid = c * _NS + s
        pltpu.sync_copy(src_hbm.at[wid], src_v)
        pltpu.sync_copy(dst_hbm.at[wid], dst_v)
        pltpu.sync_copy(zero_hbm.at[pl.ds(s * _RPT, _RPT)],
                        agg_s.at[pl.ds(s * _RPT, _RPT)])
        plsc.subcore_barrier()

        @pl.loop(0, _NCH)
        def _(ci):
            pltpu.async_copy(h_hbm.at[src_v.at[ci]], rows_v, sem).wait()
            pltpu.sync_copy(rows_v, agg_s.at[dst_v.at[ci]], add=True)

        plsc.subcore_barrier()
        pltpu.sync_copy(agg_s.at[pl.ds(s * _RPT, _RPT)],
                        out_hbm.at[c, pl.ds(s * _RPT, _RPT)])

    return k(h, src_r, dst_r, zeros)


def _mlp_body(h_ref, a_ref, w1_ref, b1_ref, w2_ref, b2_ref, o_ref):
    z = h_ref[...] + a_ref[0] + a_ref[1]
    t = jnp.dot(z, w1_ref[...], preferred_element_type=jnp.float32)
    t = jnp.maximum(t + b1_ref[...], 0.0)
    u = jnp.dot(t, w2_ref[...], preferred_element_type=jnp.float32)
    o_ref[...] = jnp.maximum(u + b2_ref[...], 0.0)


def _mlp_pool_body(h_ref, a_ref, w1_ref, b1_ref, w2_ref, b2_ref, o_ref):
    z = h_ref[...] + a_ref[0] + a_ref[1]
    t = jnp.dot(z, w1_ref[...], preferred_element_type=jnp.float32)
    t = jnp.maximum(t + b1_ref[...], 0.0)
    u = jnp.dot(t, w2_ref[...], preferred_element_type=jnp.float32)
    h2 = jnp.maximum(u + b2_ref[...], 0.0)

    @pl.when(pl.program_id(0) == 0)
    def _():
        o_ref[...] = jnp.zeros_like(o_ref)

    o_ref[...] += jnp.sum(h2, axis=0, keepdims=True) * (1.0 / _N)


_IN_SPECS = [
    pl.BlockSpec((_BR, _D), lambda i: (i, 0)),
    pl.BlockSpec((_NC, _BR, _D), lambda i: (0, i, 0)),
    pl.BlockSpec((_D, _D), lambda i: (0, 0)),
    pl.BlockSpec((1, _D), lambda i: (0, 0)),
    pl.BlockSpec((_D, _D), lambda i: (0, 0)),
    pl.BlockSpec((1, _D), lambda i: (0, 0)),
]


def _mlp_layer(h, agg, w1, b1, w2, b2):
    return pl.pallas_call(
        _mlp_body,
        grid=(_N // _BR,),
        in_specs=_IN_SPECS,
        out_specs=pl.BlockSpec((_BR, _D), lambda i: (i, 0)),
        out_shape=jax.ShapeDtypeStruct((_N, _D), jnp.float32),
    )(h, agg, w1, b1.reshape(1, _D), w2, b2.reshape(1, _D))


def _mlp_pool_layer(h, agg, w1, b1, w2, b2):
    return pl.pallas_call(
        _mlp_pool_body,
        grid=(_N // _BR,),
        in_specs=_IN_SPECS,
        out_specs=pl.BlockSpec((1, _D), lambda i: (0, 0)),
        out_shape=jax.ShapeDtypeStruct((1, _D), jnp.float32),
    )(h, agg, w1, b1.reshape(1, _D), w2, b2.reshape(1, _D))


def kernel(x, edge_index, W1_0, b1_0, W2_0, b2_0, W1_1, b1_1, W2_1, b2_1):
    src_r = edge_index[0].reshape(_NW, _NCH, _CH)
    dst_r = edge_index[1].reshape(_NW, _NCH, _CH)
    zeros = jnp.zeros((_NP, _D), jnp.float32)
    agg1 = _sc_segment_sum(x, src_r, dst_r, zeros)
    h1 = _mlp_layer(x, agg1, W1_0, b1_0, W2_0, b2_0)
    agg2 = _sc_segment_sum(h1, src_r, dst_r, zeros)
    return _mlp_pool_layer(h1, agg2, W1_1, b1_1, W2_1, b2_1)
